# Initial kernel scaffold; baseline (speedup 1.0000x reference)
import functools

import jax
import jax.numpy as jnp
from jax import lax
from jax.experimental import pallas as pl
from jax.experimental.pallas import tpu as pltpu
from jax.experimental.pallas import tpu_sc as plsc

N = 10000
E = 320000
NC, NS = 2, 16
NW = NC * NS
E_PER_TILE = E // NW
CHUNK = 80
N_CHUNKS = E_PER_TILE // CHUNK
ROWS_PER_TILE = N // NS
ZROWS = 125

_SELU_ALPHA = 1.6732632423543772
_SELU_SCALE = 1.0507009873554805


def _mesh():
    return plsc.VectorSubcoreMesh(core_axis_name="c", subcore_axis_name="s")


@functools.partial(
    pl.kernel,
    out_type=(jax.ShapeDtypeStruct((NW, N), jnp.float32),
              jax.ShapeDtypeStruct((NW, N), jnp.float32)),
    mesh=_mesh(),
    scratch_types=[
        pltpu.VMEM((E_PER_TILE,), jnp.int32),
        pltpu.VMEM((E_PER_TILE,), jnp.int32),
        pltpu.VMEM((N,), jnp.float32),
        pltpu.VMEM((N,), jnp.float32),
    ],
)
def _sc_degrees(src_hbm, dst_hbm, degs_out, degd_out, sidx, didx, hs, hd):
    wid = lax.axis_index("s") * NC + lax.axis_index("c")
    base = wid * E_PER_TILE
    pltpu.sync_copy(src_hbm.at[pl.ds(base, E_PER_TILE)], sidx)
    pltpu.sync_copy(dst_hbm.at[pl.ds(base, E_PER_TILE)], didx)
    zeros16 = jnp.zeros((16,), jnp.float32)

    @pl.loop(0, N // 16)
    def _zero(i):
        hs[pl.ds(i * 16, 16)] = zeros16
        hd[pl.ds(i * 16, 16)] = zeros16

    ones16 = jnp.ones((16,), jnp.float32)

    @pl.loop(0, E_PER_TILE // 16)
    def _acc(g):
        plsc.addupdate_scatter(hs, [sidx[pl.ds(g * 16, 16)]], ones16)
        plsc.addupdate_scatter(hd, [didx[pl.ds(g * 16, 16)]], ones16)

    pltpu.sync_copy(hs, degs_out.at[wid])
    pltpu.sync_copy(hd, degd_out.at[wid])


def _make_sc_aggregate(D):
    @functools.partial(
        pl.kernel,
        out_type=jax.ShapeDtypeStruct((NC, N, D), jnp.float32),
        mesh=_mesh(),
        scratch_types=[
            pltpu.VMEM((CHUNK,), jnp.int32),
            pltpu.VMEM((CHUNK,), jnp.int32),
            pltpu.VMEM((CHUNK, D), jnp.float32),
            pltpu.VMEM((ZROWS, D), jnp.float32),
            pltpu.VMEM_SHARED((N, D), jnp.float32),
            pltpu.SemaphoreType.DMA,
        ],
    )
    def agg(h_hbm, src_hbm, dst_hbm, out_hbm,
            sidx, didx, rows, stage, acc, sem_g):
        c = lax.axis_index("c")
        s = lax.axis_index("s")
        wid = s * NC + c
        zeros16 = jnp.zeros((16,), jnp.float32)

        @pl.loop(0, ZROWS)
        def _zstage(r):
            for j in range(D // 16):
                stage[r, pl.ds(j * 16, 16)] = zeros16

        row0 = s * ROWS_PER_TILE
        for i in range(ROWS_PER_TILE // ZROWS):
            pltpu.sync_copy(stage, acc.at[pl.ds(row0 + i * ZROWS, ZROWS)])
        plsc.subcore_barrier()

        ebase = wid * E_PER_TILE

        @pl.loop(0, N_CHUNKS)
        def _main(i):
            off = ebase + i * CHUNK
            pltpu.sync_copy(src_hbm.at[pl.ds(off, CHUNK)], sidx)
            pltpu.sync_copy(dst_hbm.at[pl.ds(off, CHUNK)], didx)
            pltpu.async_copy(h_hbm.at[sidx], rows, sem_g).wait()
            pltpu.sync_copy(rows, acc.at[didx], add=True)

        plsc.subcore_barrier()
        for i in range(ROWS_PER_TILE // ZROWS):
            r = row0 + i * ZROWS
            pltpu.sync_copy(acc.at[pl.ds(r, ZROWS)], stage)
            pltpu.sync_copy(stage, out_hbm.at[c, pl.ds(r, ZROWS)])

    return agg


_sc_agg128 = _make_sc_aggregate(128)
_sc_agg64 = _make_sc_aggregate(64)


def _tc_matmul(x, W):
    def body(x_ref, w_ref, o_ref):
        o_ref[...] = jnp.dot(x_ref[...], w_ref[...],
                             preferred_element_type=jnp.float32)

    return pl.pallas_call(
        body,
        out_shape=jax.ShapeDtypeStruct((x.shape[0], W.shape[1]), jnp.float32),
        grid=(10,),
        in_specs=[pl.BlockSpec((N // 10, x.shape[1]), lambda i: (i, 0)),
                  pl.BlockSpec((W.shape[0], W.shape[1]), lambda i: (0, 0))],
        out_specs=pl.BlockSpec((N // 10, W.shape[1]), lambda i: (i, 0)),
    )(x, W)


def _tc_norms_scale(degs_pt, degd_pt, xw1):
    def body(ds_ref, dd_ref, xw_ref, y_ref, no_ref, ni_ref):
        deg_o = jnp.sum(ds_ref[...], axis=1, keepdims=True)
        deg_i = jnp.sum(dd_ref[...], axis=1, keepdims=True)
        no = lax.rsqrt(jnp.maximum(deg_o, 1.0))
        ni = lax.rsqrt(jnp.maximum(deg_i, 1.0))
        y_ref[...] = xw_ref[...] * no
        no_ref[...] = no
        ni_ref[...] = ni

    return pl.pallas_call(
        body,
        out_shape=(jax.ShapeDtypeStruct((N, 128), jnp.float32),
                   jax.ShapeDtypeStruct((N, 1), jnp.float32),
                   jax.ShapeDtypeStruct((N, 1), jnp.float32)),
    )(degs_pt, degd_pt, xw1)


def _tc_mid(p0, p1, ni, no, b1, W2):
    def body(a_ref, b_ref, ni_ref, no_ref, b1_ref, w2_ref, o_ref):
        h = (a_ref[...] + b_ref[...]) * ni_ref[...] + b1_ref[...]
        h = _SELU_SCALE * jnp.where(h > 0, h, _SELU_ALPHA * (jnp.exp(h) - 1.0))
        y2 = h * no_ref[...]
        o_ref[...] = jnp.dot(y2, w2_ref[...],
                             preferred_element_type=jnp.float32)

    return pl.pallas_call(
        body,
        out_shape=jax.ShapeDtypeStruct((N, W2.shape[1]), jnp.float32),
    )(p0, p1, ni, no, b1, W2)


def _tc_final(p0, p1, ni, b2):
    def body(a_ref, b_ref, ni_ref, b2_ref, o_ref):
        h = (a_ref[...] + b_ref[...]) * ni_ref[...] + b2_ref[...]
        m = jnp.max(h, axis=1, keepdims=True)
        lse = jnp.log(jnp.sum(jnp.exp(h - m), axis=1, keepdims=True)) + m
        o_ref[...] = h - lse

    return pl.pallas_call(
        body,
        out_shape=jax.ShapeDtypeStruct((N, b2.shape[0]), jnp.float32),
    )(p0, p1, ni, b2)


def kernel(x, edge_index, W1, b1, W2, b2):
    src = edge_index[0].astype(jnp.int32)
    dst = edge_index[1].astype(jnp.int32)

    degs_p, degd_p = _sc_degrees(src, dst)
    xw1 = _tc_matmul(x, W1)
    y1, no, ni = _tc_norms_scale(degs_p.T, degd_p.T, xw1)

    agg1 = _sc_agg128(y1, src, dst)
    h2 = _tc_mid(agg1[0], agg1[1], ni, no, b1, W2)

    agg2 = _sc_agg64(h2, src, dst)
    return _tc_final(agg2[0], agg2[1], ni, b2)


# speedup vs baseline: 5.2010x; 5.2010x over previous
import functools

import jax
import jax.numpy as jnp
from jax import lax
from jax.experimental import pallas as pl
from jax.experimental.pallas import tpu as pltpu
from jax.experimental.pallas import tpu_sc as plsc

N = 10000
E = 320000
NC, NS = 2, 16
NW = NC * NS
E_PER_TILE = E // NW
CHUNK = 80
N_CHUNKS = E_PER_TILE // CHUNK
ROWS_PER_TILE = 624
ZROWS = 208
TAIL_ROWS = N - NS * ROWS_PER_TILE

_SELU_ALPHA = 1.6732632423543772
_SELU_SCALE = 1.0507009873554805


def _mesh():
    return plsc.VectorSubcoreMesh(core_axis_name="c", subcore_axis_name="s")


@functools.partial(
    pl.kernel,
    out_type=(jax.ShapeDtypeStruct((NW, N), jnp.float32),
              jax.ShapeDtypeStruct((NW, N), jnp.float32)),
    mesh=_mesh(),
    scratch_types=[
        pltpu.VMEM((E_PER_TILE,), jnp.int32),
        pltpu.VMEM((E_PER_TILE,), jnp.int32),
        pltpu.VMEM((N,), jnp.float32),
        pltpu.VMEM((N,), jnp.float32),
    ],
    compiler_params=pltpu.CompilerParams(needs_layout_passes=False),
)
def _sc_degrees(src_hbm, dst_hbm, degs_out, degd_out, sidx, didx, hs, hd):
    wid = lax.axis_index("s") * NC + lax.axis_index("c")
    base = wid * E_PER_TILE
    pltpu.sync_copy(src_hbm.at[pl.ds(base, E_PER_TILE)], sidx)
    pltpu.sync_copy(dst_hbm.at[pl.ds(base, E_PER_TILE)], didx)
    zeros16 = jnp.zeros((16,), jnp.float32)

    @pl.loop(0, N // 16)
    def _zero(i):
        hs[pl.ds(i * 16, 16)] = zeros16
        hd[pl.ds(i * 16, 16)] = zeros16

    @pl.loop(0, E_PER_TILE // 16)
    def _acc(g):
        si = sidx[pl.ds(g * 16, 16)]
        cnt_s, last_s = plsc.scan_count(si)
        plsc.addupdate_scatter(hs, [si], cnt_s.astype(jnp.float32),
                               mask=last_s)
        di = didx[pl.ds(g * 16, 16)]
        cnt_d, last_d = plsc.scan_count(di)
        plsc.addupdate_scatter(hd, [di], cnt_d.astype(jnp.float32),
                               mask=last_d)

    pltpu.sync_copy(hs, degs_out.at[wid])
    pltpu.sync_copy(hd, degd_out.at[wid])


def _make_sc_aggregate(D):
    @functools.partial(
        pl.kernel,
        out_type=jax.ShapeDtypeStruct((NC, N, D), jnp.float32),
        mesh=_mesh(),
        compiler_params=(None if D == 128 else
                         pltpu.CompilerParams(use_tc_tiling_on_sc=False)),
        scratch_types=[
            pltpu.VMEM((CHUNK,), jnp.int32),
            pltpu.VMEM((CHUNK,), jnp.int32),
            pltpu.VMEM((CHUNK, D), jnp.float32),
            pltpu.VMEM((ZROWS, D), jnp.float32),
            pltpu.VMEM_SHARED((N, D), jnp.float32),
            pltpu.SemaphoreType.DMA,
        ],
    )
    def agg(h_hbm, src_hbm, dst_hbm, out_hbm,
            sidx, didx, rows, stage, acc, sem_g):
        c = lax.axis_index("c")
        s = lax.axis_index("s")
        wid = s * NC + c
        zeros16 = jnp.zeros((16,), jnp.float32)

        @pl.loop(0, ZROWS)
        def _zstage(r):
            for j in range(D // 16):
                stage[r, pl.ds(j * 16, 16)] = zeros16

        row0 = s * ROWS_PER_TILE
        for i in range(ROWS_PER_TILE // ZROWS):
            pltpu.sync_copy(stage, acc.at[pl.ds(row0 + i * ZROWS, ZROWS)])

        @pl.when(s == NS - 1)
        def _ztail():
            pltpu.sync_copy(stage.at[pl.ds(0, TAIL_ROWS)],
                            acc.at[pl.ds(NS * ROWS_PER_TILE, TAIL_ROWS)])

        plsc.subcore_barrier()

        ebase = wid * E_PER_TILE

        @pl.loop(0, N_CHUNKS)
        def _main(i):
            off = ebase + i * CHUNK
            pltpu.sync_copy(src_hbm.at[pl.ds(off, CHUNK)], sidx)
            pltpu.sync_copy(dst_hbm.at[pl.ds(off, CHUNK)], didx)
            pltpu.async_copy(h_hbm.at[sidx], rows, sem_g).wait()
            pltpu.sync_copy(rows, acc.at[didx], add=True)

        plsc.subcore_barrier()
        for i in range(ROWS_PER_TILE // ZROWS):
            r = row0 + i * ZROWS
            pltpu.sync_copy(acc.at[pl.ds(r, ZROWS)], stage)
            pltpu.sync_copy(stage, out_hbm.at[c, pl.ds(r, ZROWS)])

        @pl.when(s == NS - 1)
        def _wtail():
            r = NS * ROWS_PER_TILE
            pltpu.sync_copy(acc.at[pl.ds(r, TAIL_ROWS)],
                            stage.at[pl.ds(0, TAIL_ROWS)])
            pltpu.sync_copy(stage.at[pl.ds(0, TAIL_ROWS)],
                            out_hbm.at[c, pl.ds(r, TAIL_ROWS)])

    return agg


_sc_agg128 = _make_sc_aggregate(128)
_sc_agg64 = _make_sc_aggregate(64)


def _tc_matmul(x, W):
    def body(x_ref, w_ref, o_ref):
        o_ref[...] = jnp.dot(x_ref[...], w_ref[...],
                             preferred_element_type=jnp.float32)

    return pl.pallas_call(
        body,
        out_shape=jax.ShapeDtypeStruct((x.shape[0], W.shape[1]), jnp.float32),
        grid=(10,),
        in_specs=[pl.BlockSpec((N // 10, x.shape[1]), lambda i: (i, 0)),
                  pl.BlockSpec((W.shape[0], W.shape[1]), lambda i: (0, 0))],
        out_specs=pl.BlockSpec((N // 10, W.shape[1]), lambda i: (i, 0)),
    )(x, W)


def _tc_norms_scale(degs_pt, degd_pt, xw1):
    def body(ds_ref, dd_ref, xw_ref, y_ref, no_ref, ni_ref):
        deg_o = jnp.sum(ds_ref[...], axis=1, keepdims=True)
        deg_i = jnp.sum(dd_ref[...], axis=1, keepdims=True)
        no = lax.rsqrt(jnp.maximum(deg_o, 1.0))
        ni = lax.rsqrt(jnp.maximum(deg_i, 1.0))
        y_ref[...] = xw_ref[...] * no
        no_ref[...] = no
        ni_ref[...] = ni

    return pl.pallas_call(
        body,
        out_shape=(jax.ShapeDtypeStruct((N, 128), jnp.float32),
                   jax.ShapeDtypeStruct((N, 1), jnp.float32),
                   jax.ShapeDtypeStruct((N, 1), jnp.float32)),
    )(degs_pt, degd_pt, xw1)


def _tc_mid(p0, p1, ni, no, b1, W2):
    def body(a_ref, b_ref, ni_ref, no_ref, b1_ref, w2_ref, o_ref):
        h = (a_ref[...] + b_ref[...]) * ni_ref[...] + b1_ref[...]
        h = _SELU_SCALE * jnp.where(h > 0, h, _SELU_ALPHA * (jnp.exp(h) - 1.0))
        y2 = h * no_ref[...]
        o_ref[...] = jnp.dot(y2, w2_ref[...],
                             preferred_element_type=jnp.float32)

    return pl.pallas_call(
        body,
        out_shape=jax.ShapeDtypeStruct((N, W2.shape[1]), jnp.float32),
    )(p0, p1, ni, no, b1, W2)


def _tc_final(p0, p1, ni, b2):
    def body(a_ref, b_ref, ni_ref, b2_ref, o_ref):
        h = (a_ref[...] + b_ref[...]) * ni_ref[...] + b2_ref[...]
        m = jnp.max(h, axis=1, keepdims=True)
        lse = jnp.log(jnp.sum(jnp.exp(h - m), axis=1, keepdims=True)) + m
        o_ref[...] = h - lse

    return pl.pallas_call(
        body,
        out_shape=jax.ShapeDtypeStruct((N, b2.shape[0]), jnp.float32),
    )(p0, p1, ni, b2)


def kernel(x, edge_index, W1, b1, W2, b2):
    src = edge_index[0].astype(jnp.int32)
    dst = edge_index[1].astype(jnp.int32)

    degs_p, degd_p = _sc_degrees(src, dst)
    xw1 = _tc_matmul(x, W1)
    y1, no, ni = _tc_norms_scale(degs_p.T, degd_p.T, xw1)

    agg1 = _sc_agg128(y1, src, dst)
    h2 = _tc_mid(agg1[0], agg1[1], ni, no, b1, W2)

    agg2 = _sc_agg64(h2, src, dst)
    return _tc_final(agg2[0], agg2[1], ni, b2)
